# baseline (speedup 1.0000x reference)
import functools
import math

import jax
import jax.numpy as jnp
from jax import lax
from jax.experimental import pallas as pl
from jax.experimental.pallas import tpu as pltpu
from jax.experimental.pallas import tpu_sc as plsc

N0 = 10000
E0 = 160000
DIM_IN = 128
HID = 512
HEADS = 2
RATIO = 0.7

NC = 2
NS = 16
NW = NC * NS
E_PAD = 160256
EPW = E_PAD // NW
BM = 512

_NS_LIST = [N0]
for _ in range(4):
    _NS_LIST.append(int(math.ceil(RATIO * _NS_LIST[-1])))


def _rup(v, m):
    return ((v + m - 1) // m) * m


_NPAD = [_rup(v + 1, BM) for v in _NS_LIST]

_MESH = plsc.VectorSubcoreMesh(core_axis_name="c", subcore_axis_name="s")


def _lane16():
    return lax.broadcasted_iota(jnp.int32, (16,), 0)


def _take16(x, idx):
    return lax.gather(
        x, idx[:, None],
        dimension_numbers=lax.GatherDimensionNumbers(
            offset_dims=(), collapsed_slice_dims=(0,), start_index_map=(0,)),
        slice_sizes=(1,),
        mode=lax.GatherScatterMode.PROMISE_IN_BOUNDS)


def _bsum16(x):
    lane = _lane16()
    for d in (1, 2, 4, 8):
        x = x + _take16(x, jnp.bitwise_xor(lane, d))
    return x


def _bcast16(x, j):
    return _take16(x, jnp.full((16,), 1, jnp.int32) * j)


def _pfx16(x):
    lane = _lane16()
    for d in (1, 2, 4, 8):
        sh = _take16(x, jnp.maximum(lane - d, 0))
        x = x + jnp.where(lane >= d, sh, 0)
    return x


def _mm(x, w, b3, nvalid, nch, npad, din):
    bk = min(din, 512)
    kb = din // bk
    mb = npad // BM

    def body(x_ref, w_ref, b_ref, o_ref):
        i = pl.program_id(0)
        kk = pl.program_id(2)
        contrib = jnp.dot(x_ref[...], w_ref[...],
                          preferred_element_type=jnp.float32)[None]
        prev = jnp.where(kk == 0, 0.0, o_ref[...])
        acc = prev + contrib

        @pl.when(kk < kb - 1)
        def _():
            o_ref[...] = acc

        @pl.when(kk == kb - 1)
        def _():
            rows = lax.broadcasted_iota(jnp.int32, (1, BM, 128), 1) + i * BM
            val = acc + b_ref[...]
            o_ref[...] = jnp.where(rows < nvalid, val, 0.0)

    return pl.pallas_call(
        body,
        grid=(mb, nch, kb),
        in_specs=[
            pl.BlockSpec((BM, bk), lambda i, j, kk: (i, kk)),
            pl.BlockSpec((bk, 128), lambda i, j, kk: (kk, j)),
            pl.BlockSpec((1, 1, 128), lambda i, j, kk: (j, 0, 0)),
        ],
        out_specs=pl.BlockSpec((1, BM, 128), lambda i, j, kk: (j, i, 0)),
        out_shape=jax.ShapeDtypeStruct((nch, npad, 128), jnp.float32),
    )(x, w, b3)


def _maxexp(logit2d):
    rows_per_head = E_PAD // 128

    def body(l_ref, o_ref):
        x = l_ref[...]
        r = lax.broadcasted_iota(jnp.int32, x.shape, 0)
        h0 = r < rows_per_head
        m0 = jnp.max(jnp.where(h0, x, -jnp.inf))
        m1 = jnp.max(jnp.where(h0, -jnp.inf, x))
        o_ref[...] = jnp.exp(x - jnp.where(h0, m0, m1))

    return pl.pallas_call(
        body,
        out_shape=jax.ShapeDtypeStruct(logit2d.shape, jnp.float32),
    )(logit2d)


def _combine_gat(acc4d, b4, nvalid, npad):
    mb = npad // BM

    def body(a_ref, b_ref, h_ref, c_ref):
        i = pl.program_id(0)
        a = a_ref[...]
        s0 = a[0, 8] + a[1, 8] + 1e-16
        s1 = a[0, 9] + a[1, 9] + 1e-16
        rows = lax.broadcasted_iota(jnp.int32, (BM, 128), 0) + i * BM
        ok = rows < nvalid
        for cc in range(4):
            h0 = (a[0, cc] + a[1, cc]) / s0
            h1 = (a[0, 4 + cc] + a[1, 4 + cc]) / s1
            val = (h0 + h1) * 0.5 + b_ref[cc]
            val = jnp.where(ok, jnp.maximum(val, 0.0), 0.0)
            h_ref[:, cc * 128:(cc + 1) * 128] = val
            c_ref[cc] = val

    return pl.pallas_call(
        body,
        grid=(mb,),
        in_specs=[
            pl.BlockSpec((2, 10, BM, 128), lambda i: (0, 0, i, 0)),
            pl.BlockSpec((4, 1, 128), lambda i: (0, 0, 0)),
        ],
        out_specs=[
            pl.BlockSpec((BM, 512), lambda i: (i, 0)),
            pl.BlockSpec((4, BM, 128), lambda i: (0, i, 0)),
        ],
        out_shape=[
            jax.ShapeDtypeStruct((npad, 512), jnp.float32),
            jax.ShapeDtypeStruct((4, npad, 128), jnp.float32),
        ],
    )(acc4d, b4)


def _combine_sag(acc4d, npad):
    mb = npad // BM

    def body(a_ref, o_ref):
        a = a_ref[...]
        for cc in range(4):
            o_ref[:, cc * 128:(cc + 1) * 128] = a[0, cc] + a[1, cc]

    return pl.pallas_call(
        body,
        grid=(mb,),
        in_specs=[pl.BlockSpec((2, 4, BM, 128), lambda i: (0, 0, i, 0))],
        out_specs=pl.BlockSpec((BM, 512), lambda i: (i, 0)),
        out_shape=jax.ShapeDtypeStruct((npad, 512), jnp.float32),
    )(acc4d)


def _thresh(screp, k, nvalid, npad):
    def body(s_ref, lo_ref):
        x = s_ref[...]
        ri = lax.broadcasted_iota(jnp.int32, x.shape, 0)
        ci = lax.broadcasted_iota(jnp.int32, x.shape, 1)
        ok = ri < nvalid
        cnt = ok & (ci == 0)
        xm = jnp.where(ok, x, -jnp.inf)
        lo0 = jnp.min(jnp.where(ok, x, jnp.inf))
        hi0 = jnp.max(xm)

        def bis(_, lohi):
            lo, hi = lohi
            mid = 0.5 * (lo + hi)
            c = jnp.sum(jnp.where(cnt & (x >= mid), 1, 0))
            good = c >= k
            return jnp.where(good, mid, lo), jnp.where(good, hi, mid)

        lo, hi = lax.fori_loop(0, 64, bis, (lo0, hi0))
        lo_ref[...] = jnp.full((8, 128), lo)

    return pl.pallas_call(
        body,
        out_shape=jax.ShapeDtypeStruct((8, 128), jnp.float32),
    )(screp)


def _scale_h(h2d, screp, npad):
    mb = npad // BM

    def body(h_ref, s_ref, o_ref):
        t = jnp.tanh(s_ref[...])
        for cc in range(4):
            o_ref[:, cc * 128:(cc + 1) * 128] = \
                h_ref[:, cc * 128:(cc + 1) * 128] * t

    return pl.pallas_call(
        body,
        grid=(mb,),
        in_specs=[
            pl.BlockSpec((BM, 512), lambda i: (i, 0)),
            pl.BlockSpec((BM, 128), lambda i: (i, 0)),
        ],
        out_specs=pl.BlockSpec((BM, 512), lambda i: (i, 0)),
        out_shape=jax.ShapeDtypeStruct((npad, 512), jnp.float32),
    )(h2d, screp)


def _pool_stats(xn, k, npadn):
    mb = npadn // BM

    def body(x_ref, o_ref):
        i = pl.program_id(0)
        x = x_ref[...]
        rows = lax.broadcasted_iota(jnp.int32, (BM, 512), 0) + i * BM
        ok = rows < k

        @pl.when(i == 0)
        def _():
            ri = lax.broadcasted_iota(jnp.int32, (8, 512), 0)
            o_ref[...] = jnp.where(ri == 0, -jnp.inf, 0.0)

        mx = jnp.max(jnp.where(ok, x, -jnp.inf), axis=0)
        sm = jnp.sum(jnp.where(ok, x, 0.0), axis=0)
        o_ref[0, :] = jnp.maximum(o_ref[0, :], mx)
        o_ref[1, :] = o_ref[1, :] + sm

        @pl.when(i == mb - 1)
        def _():
            o_ref[1, :] = o_ref[1, :] * (1.0 / k)

    return pl.pallas_call(
        body,
        grid=(mb,),
        in_specs=[pl.BlockSpec((BM, 512), lambda i: (i, 0))],
        out_specs=pl.BlockSpec((8, 512), lambda i: (0, 0)),
        out_shape=jax.ShapeDtypeStruct((8, 512), jnp.float32),
    )(xn)


def _head(s0, s1, s2, s3, w1, b1, w2, b2, w3p, b3p):
    def body(s0r, s1r, s2r, s3r, w1r, b1r, w2r, b2r, w3r, b3r, lo_ref, pr_ref):
        mx = s0r[0, :] + s1r[0, :] + s2r[0, :] + s3r[0, :]
        mn = s0r[1, :] + s1r[1, :] + s2r[1, :] + s3r[1, :]
        z = jnp.concatenate([mx, mn])[None]
        z8 = jnp.broadcast_to(z, (8, 1024))
        z1 = jnp.maximum(jnp.dot(z8, w1r[...],
                                 preferred_element_type=jnp.float32)
                         + b1r[0][None], 0.0)
        z2 = jnp.maximum(jnp.dot(z1, w2r[...],
                                 preferred_element_type=jnp.float32)
                         + b2r[0][None], 0.0)
        lg = jnp.dot(z2, w3r[...], preferred_element_type=jnp.float32) \
            + b3r[0][None]
        lo_ref[...] = lg
        ci = lax.broadcasted_iota(jnp.int32, (8, 128), 1)
        okc = ci < 5
        m = jnp.max(jnp.where(okc, lg, -jnp.inf), axis=1, keepdims=True)
        e = jnp.where(okc, jnp.exp(lg - m), 0.0)
        pr_ref[...] = e / jnp.sum(e, axis=1, keepdims=True)

    return pl.pallas_call(
        body,
        out_shape=[
            jax.ShapeDtypeStruct((8, 128), jnp.float32),
            jax.ShapeDtypeStruct((8, 128), jnp.float32),
        ],
    )(s0, s1, s2, s3, w1, b1, w2, b2, w3p, b3p)


def _sc_logits(npad):
    nbatch = EPW // 16

    @functools.partial(
        pl.kernel,
        mesh=_MESH,
        out_type=jax.ShapeDtypeStruct((2 * E_PAD,), jnp.float32),
        scratch_types=[
            pltpu.VMEM((EPW,), jnp.int32),
            pltpu.VMEM((EPW,), jnp.int32),
            pltpu.VMEM((1024,), jnp.float32),
            pltpu.VMEM((256, 128), jnp.float32),
            pltpu.VMEM((EPW,), jnp.float32),
            pltpu.VMEM((EPW,), jnp.float32),
            pltpu.SemaphoreType.DMA,
        ],
    )
    def kern(proj_hbm, src_hbm, dst_hbm, att_hbm, out_hbm,
             src_v, dst_v, att_v, rows_v, lg0_v, lg1_v, sem):
        wid = lax.axis_index("s") * NC + lax.axis_index("c")
        base = wid * EPW
        pltpu.sync_copy(src_hbm.at[pl.ds(base, EPW)], src_v)
        pltpu.sync_copy(dst_hbm.at[pl.ds(base, EPW)], dst_v)
        pltpu.sync_copy(att_hbm, att_v)
        lane = _lane16()

        def batch(bi, _):
            sidx = src_v[pl.ds(bi * 16, 16)]
            didx = dst_v[pl.ds(bi * 16, 16)]
            handles = []
            for c in range(8):
                handles.append(pltpu.async_copy(
                    proj_hbm.at[sidx + c * npad],
                    rows_v.at[pl.ds(c * 16, 16)], sem))
            for c in range(8):
                handles.append(pltpu.async_copy(
                    proj_hbm.at[didx + (8 + c) * npad],
                    rows_v.at[pl.ds((8 + c) * 16, 16)], sem))
            for h in handles:
                h.wait()

            def edge(j, carry):
                lg0c, lg1c = carry
                acc0 = jnp.zeros((16,), jnp.float32)
                acc1 = jnp.zeros((16,), jnp.float32)
                for c in range(8):
                    for v in range(8):
                        rl = rows_v[c * 16 + j, pl.ds(v * 16, 16)]
                        rr = rows_v[(8 + c) * 16 + j, pl.ds(v * 16, 16)]
                        t = rl + rr
                        t = jnp.where(t >= 0, t, 0.2 * t)
                        av = att_v[pl.ds(c * 128 + v * 16, 16)]
                        if c < 4:
                            acc0 = acc0 + t * av
                        else:
                            acc1 = acc1 + t * av
                lg0c = jnp.where(lane == j, _bsum16(acc0), lg0c)
                lg1c = jnp.where(lane == j, _bsum16(acc1), lg1c)
                return (lg0c, lg1c)

            lg0, lg1 = lax.fori_loop(
                0, 16, edge,
                (jnp.zeros((16,), jnp.float32), jnp.zeros((16,), jnp.float32)))
            lg0_v[pl.ds(bi * 16, 16)] = lg0
            lg1_v[pl.ds(bi * 16, 16)] = lg1
            return 0

        lax.fori_loop(0, nbatch, batch, 0)
        pltpu.sync_copy(lg0_v, out_hbm.at[pl.ds(base, EPW)])
        pltpu.sync_copy(lg1_v, out_hbm.at[pl.ds(E_PAD + base, EPW)])

    return kern


def _sc_agg(npad, nch, scale_rows):
    nbatch = EPW // 16
    srows = npad // NS

    scratch = [
        pltpu.VMEM((EPW,), jnp.int32),
        pltpu.VMEM((EPW,), jnp.int32),
        pltpu.VMEM((16, 128), jnp.float32),
        pltpu.VMEM((16, 128), jnp.float32),
        pltpu.VMEM((32, 128), jnp.float32),
        pltpu.VMEM_SHARED((npad, 128), jnp.float32),
        pltpu.SemaphoreType.DMA,
    ]
    if scale_rows is not None:
        scratch.insert(2, pltpu.VMEM((EPW,), jnp.float32))

    def body(tbl_hbm, scale_hbm, src_hbm, dst_hbm, out_hbm,
             src_v, dst_v, esl_v, rows_v, sca_v, zb_v, spt, sem):
        cid = lax.axis_index("c")
        sid = lax.axis_index("s")
        wid = sid * NC + cid
        base = wid * EPW
        sb = sid * srows
        lane = _lane16()
        pltpu.sync_copy(src_hbm.at[pl.ds(base, EPW)], src_v)
        pltpu.sync_copy(dst_hbm.at[pl.ds(base, EPW)], dst_v)

        def zrow(r, _):
            for v in range(8):
                zb_v[r, pl.ds(v * 16, 16)] = jnp.zeros((16,), jnp.float32)
            return 0

        lax.fori_loop(0, 32, zrow, 0)

        for c in range(nch):
            for t in range(srows // 32):
                pltpu.sync_copy(zb_v, spt.at[pl.ds(sb + t * 32, 32)])
            plsc.subcore_barrier()
            if scale_rows is not None:
                pltpu.sync_copy(
                    scale_hbm.at[pl.ds(scale_rows[c] * E_PAD + base, EPW)],
                    esl_v)

            def batch(bi, _):
                sidx = src_v[pl.ds(bi * 16, 16)]
                didx = dst_v[pl.ds(bi * 16, 16)]
                pltpu.async_copy(tbl_hbm.at[sidx + c * npad],
                                 rows_v, sem).wait()
                if scale_rows is not None:
                    evb = esl_v[pl.ds(bi * 16, 16)]

                    def srow(j, _):
                        ev = _bcast16(evb, j)
                        for v in range(8):
                            sca_v[j, pl.ds(v * 16, 16)] = \
                                rows_v[j, pl.ds(v * 16, 16)] * ev
                        return 0
                    lax.fori_loop(0, 16, srow, 0)
                    pltpu.sync_copy(sca_v, spt.at[didx], add=True)
                else:
                    pltpu.sync_copy(rows_v, spt.at[didx], add=True)
                return 0

            lax.fori_loop(0, nbatch, batch, 0)
            plsc.subcore_barrier()
            pltpu.sync_copy(
                spt.at[pl.ds(sb, srows)],
                out_hbm.at[pl.ds((cid * nch + c) * npad + sb, srows)])
            plsc.subcore_barrier()

    out_t = jax.ShapeDtypeStruct((2 * nch * npad, 128), jnp.float32)
    if scale_rows is None:
        def body2(tbl_hbm, src_hbm, dst_hbm, out_hbm,
                  src_v, dst_v, rows_v, sca_v, zb_v, spt, sem):
            body(tbl_hbm, None, src_hbm, dst_hbm, out_hbm,
                 src_v, dst_v, None, rows_v, sca_v, zb_v, spt, sem)
        return functools.partial(pl.kernel, mesh=_MESH, out_type=out_t,
                                 scratch_types=scratch)(body2)
    return functools.partial(pl.kernel, mesh=_MESH, out_type=out_t,
                             scratch_types=scratch)(body)


def _sc_compact(npad, k, nvalid):
    nchunk = npad // 16

    @functools.partial(
        pl.kernel,
        mesh=_MESH,
        out_type=jax.ShapeDtypeStruct((npad,), jnp.int32),
        scratch_types=[
            pltpu.VMEM((npad,), jnp.float32),
            pltpu.VMEM((16,), jnp.float32),
            pltpu.VMEM((npad,), jnp.int32),
        ],
    )
    def kern(score_hbm, lo_hbm, nmap_hbm, sc_v, lo_v, nm_v):
        wid = lax.axis_index("s") * NC + lax.axis_index("c")

        @pl.when(wid == 0)
        def _():
            pltpu.sync_copy(score_hbm, sc_v)
            pltpu.sync_copy(lo_hbm.at[pl.ds(0, 16)], lo_v)
            lov = lo_v[...]
            lane = _lane16()

            def chunk(i, carry):
                fid = lane + i * 16
                sv = sc_v[pl.ds(i * 16, 16)]
                m = (sv >= lov) & (fid < nvalid)
                mi = jnp.where(m, 1, 0)
                cs = _pfx16(mi) + carry
                sel = m & (cs <= k)
                nm_v[pl.ds(i * 16, 16)] = jnp.where(sel, cs - 1, -1)
                return _bcast16(cs, 15)

            lax.fori_loop(0, nchunk, chunk, jnp.zeros((16,), jnp.int32))
            pltpu.sync_copy(nm_v, nmap_hbm)

    return kern


def _sc_scatter_rows(npad, npadn, k):
    rpw = npad // NW
    nbatch = rpw // 16

    @functools.partial(
        pl.kernel,
        mesh=_MESH,
        out_type=jax.ShapeDtypeStruct((npadn, 512), jnp.float32),
        scratch_types=[
            pltpu.VMEM((rpw,), jnp.int32),
            pltpu.VMEM((16, 512), jnp.float32),
            pltpu.SemaphoreType.DMA,
        ],
    )
    def kern(h_hbm, nmap_hbm, xn_hbm, nm_v, rows_v, sem):
        wid = lax.axis_index("s") * NC + lax.axis_index("c")
        base = wid * rpw
        pltpu.sync_copy(nmap_hbm.at[pl.ds(base, rpw)], nm_v)
        kv = jnp.full((16,), k, jnp.int32)

        def batch(bi, _):
            nm = nm_v[pl.ds(bi * 16, 16)]
            nms = jnp.where(nm >= 0, nm, kv)
            pltpu.sync_copy(h_hbm.at[pl.ds(base + bi * 16, 16)], rows_v)
            pltpu.async_copy(rows_v, xn_hbm.at[nms], sem).wait()
            return 0

        lax.fori_loop(0, nbatch, batch, 0)

    return kern


def _sc_remap(npad, k):
    nbatch = EPW // 16

    @functools.partial(
        pl.kernel,
        mesh=_MESH,
        out_type=[
            jax.ShapeDtypeStruct((E_PAD,), jnp.int32),
            jax.ShapeDtypeStruct((E_PAD,), jnp.int32),
        ],
        scratch_types=[
            pltpu.VMEM((EPW,), jnp.int32),
            pltpu.VMEM((EPW,), jnp.int32),
            pltpu.VMEM((EPW,), jnp.int32),
            pltpu.VMEM((EPW,), jnp.int32),
            pltpu.VMEM((16, 128), jnp.int32),
            pltpu.VMEM((16, 128), jnp.int32),
            pltpu.SemaphoreType.DMA,
        ],
    )
    def kern(nmap_hbm, src_hbm, dst_hbm, ns_hbm, nd_hbm,
             src_v, dst_v, ns_v, nd_v, nrs_v, nrd_v, sem):
        wid = lax.axis_index("s") * NC + lax.axis_index("c")
        base = wid * EPW
        pltpu.sync_copy(src_hbm.at[pl.ds(base, EPW)], src_v)
        pltpu.sync_copy(dst_hbm.at[pl.ds(base, EPW)], dst_v)
        kv = jnp.full((16,), k, jnp.int32)
        lane = _lane16()
        zi = jnp.zeros((16,), jnp.int32)

        def chunk(i, _):
            s = src_v[pl.ds(i * 16, 16)]
            d = dst_v[pl.ds(i * 16, 16)]
            h1 = pltpu.async_copy(nmap_hbm.at[s], nrs_v, sem)
            h2 = pltpu.async_copy(nmap_hbm.at[d], nrd_v, sem)
            h1.wait()
            h2.wait()

            def ext(j, c):
                ns_, nd_ = c
                ns_ = jnp.where(lane == j, nrs_v[j, pl.ds(0, 16)], ns_)
                nd_ = jnp.where(lane == j, nrd_v[j, pl.ds(0, 16)], nd_)
                return (ns_, nd_)

            ns, nd = lax.fori_loop(0, 16, ext, (zi, zi))
            ok = (ns >= 0) & (nd >= 0)
            ns_v[pl.ds(i * 16, 16)] = jnp.where(ok, ns, kv)
            nd_v[pl.ds(i * 16, 16)] = jnp.where(ok, nd, kv)
            return 0

        lax.fori_loop(0, nbatch, chunk, 0)
        pltpu.sync_copy(ns_v, ns_hbm.at[pl.ds(base, EPW)])
        pltpu.sync_copy(nd_v, nd_hbm.at[pl.ds(base, EPW)])

    return kern


_GAT_SCALE_ROWS = [0, 0, 0, 0, 1, 1, 1, 1, 0, 1]


def kernel(x, edge_index, batch, params):
    p = params
    src = jnp.concatenate(
        [edge_index[0], jnp.full((E_PAD - E0,), N0, jnp.int32)])
    dst = jnp.concatenate(
        [edge_index[1], jnp.full((E_PAD - E0,), N0, jnp.int32)])
    h2d = jnp.zeros((_NPAD[0], DIM_IN), jnp.float32).at[:N0].set(x)

    stats = []
    for l in range(4):
        n, npad, k, npadn = _NS_LIST[l], _NPAD[l], _NS_LIST[l + 1], _NPAD[l + 1]
        din = DIM_IN if l == 0 else HID
        wcat = jnp.concatenate([p['gat%d_Wl' % l], p['gat%d_Wr' % l]], axis=1)
        bcat = jnp.concatenate(
            [p['gat%d_bl' % l], p['gat%d_br' % l]]).reshape(16, 1, 128)
        proj = _mm(h2d, wcat, bcat, n, 16, npad, din)
        proj2d = proj.reshape(16 * npad, 128)
        att_flat = p['gat%d_att' % l].reshape(1024)
        logit = _sc_logits(npad)(proj2d, src, dst, att_flat)
        e2d = _maxexp(logit.reshape(2 * E_PAD // 128, 128))
        e_flat = e2d.reshape(2 * E_PAD)
        gat_tbl = jnp.concatenate(
            [proj2d[:8 * npad], jnp.ones((2 * npad, 128), jnp.float32)])
        acc = _sc_agg(npad, 10, _GAT_SCALE_ROWS)(gat_tbl, e_flat, src, dst)
        b4 = p['gat%d_b' % l].reshape(4, 1, 128)
        h2d, hch = _combine_gat(acc.reshape(2, 10, npad, 128), b4, n, npad)
        agg_acc = _sc_agg(npad, 4, None)(
            hch.reshape(4 * npad, 128), src, dst)
        agg2d = _combine_sag(agg_acc.reshape(2, 4, npad, 128), npad)
        cat = jnp.concatenate([agg2d, h2d], axis=1)
        wsc = jnp.tile(
            jnp.concatenate([p['pool%d_Wrel' % l], p['pool%d_Wroot' % l]]),
            (1, 128))
        bsc = jnp.full((1, 1, 128), p['pool%d_brel' % l][0], jnp.float32)
        screp = _mm(cat, wsc, bsc, n, 1, npad, 2 * HID)[0]
        score = screp[:, 0]
        lo8 = _thresh(screp, k, n, npad)
        nmap = _sc_compact(npad, k, n)(score, lo8[0])
        hs = _scale_h(h2d, screp, npad)
        xn = _sc_scatter_rows(npad, npadn, k)(hs, nmap)
        nmap128 = jnp.broadcast_to(nmap[:, None], (npad, 128))
        src, dst = _sc_remap(npad, k)(nmap128, src, dst)
        stats.append(_pool_stats(xn, k, npadn))
        h2d = xn

    w3p = jnp.pad(p['lin3_W'], ((0, 0), (0, 123)))
    b3p = jnp.pad(p['lin3_b'], (0, 123)).reshape(1, 128)
    lg, pr = _head(stats[0], stats[1], stats[2], stats[3],
                   p['lin1_W'], p['lin1_b'].reshape(1, 512),
                   p['lin2_W'], p['lin2_b'].reshape(1, 256),
                   w3p, b3p)
    return lg[0:1, 0:5], pr[0:1, 0:5]
